# baseline (speedup 1.0000x reference)
import functools

import jax
import jax.numpy as jnp
from jax import lax
from jax.experimental import pallas as pl
from jax.experimental.pallas import tpu as pltpu
from jax.experimental.pallas import tpu_sc as plsc

B, T, D = 1024, 50, 64
V_ANS = 1000
V_TAB = V_ANS + 2 * T
NC, NS, L = 2, 16, 16
NW = NC * NS
BPW = B // NW
NPAIR = BPW // 2
ANS_R = V_ANS * D // 128
OCR_R = T * D // 128


@functools.partial(
    pl.kernel,
    mesh=plsc.VectorSubcoreMesh(core_axis_name="c", subcore_axis_name="s"),
    out_type=jax.ShapeDtypeStruct((B * T * D // 128, 128), jnp.float32),
    scratch_types=[
        pltpu.VMEM((V_TAB * D // 128, 128), jnp.float32),
        pltpu.VMEM((BPW, T), jnp.int32),
        pltpu.VMEM((T * D // 128, 128), jnp.float32),
        pltpu.VMEM((T * D // 128, 128), jnp.float32),
        pltpu.SemaphoreType.DMA,
        pltpu.SemaphoreType.DMA,
        pltpu.SemaphoreType.DMA,
        pltpu.SemaphoreType.DMA,
        pltpu.SemaphoreType.DMA,
        pltpu.SemaphoreType.DMA,
    ],
    compiler_params=pltpu.CompilerParams(
        needs_layout_passes=False, use_tc_tiling_on_sc=False
    ),
)
def _gather_kernel(
    ans_hbm, ocr_hbm, inds_hbm, out_hbm,
    table, idx_all, out0, out1,
    sem_ans, sem_idx, so0, so1, su0, su1,
):
    wid = lax.axis_index("s") * NC + lax.axis_index("c")
    b0 = wid * BPW

    cp_ans = pltpu.async_copy(ans_hbm, table.at[pl.ds(0, ANS_R)], sem_ans)
    cp_idx = pltpu.async_copy(inds_hbm.at[pl.ds(b0, BPW)], idx_all, sem_idx)
    pltpu.async_copy(
        ocr_hbm.at[pl.ds(b0 * OCR_R, OCR_R)], table.at[pl.ds(ANS_R, OCR_R)], so0
    )
    pltpu.async_copy(
        ocr_hbm.at[pl.ds((b0 + 1) * OCR_R, OCR_R)],
        table.at[pl.ds(ANS_R + OCR_R, OCR_R)], so1,
    )
    cp_idx.wait()
    cp_ans.wait()

    def do_batch(j, i, slot, out_buf, sem_o, sem_u):
        slot_ds = pl.ds(ANS_R + OCR_R * slot, OCR_R)
        pltpu.make_async_copy(
            ocr_hbm.at[pl.ds(0, OCR_R)], table.at[slot_ds], sem_o
        ).wait()

        @pl.when(j > 0)
        def _():
            pltpu.make_async_copy(
                out_buf, out_hbm.at[pl.ds(0, OCR_R)], sem_u
            ).wait()

        for r in range(T):
            row = plsc.load_gather(
                idx_all,
                [jnp.full((L,), i, jnp.int32), jnp.full((L,), r, jnp.int32)],
            )
            if slot == 1:
                row = jnp.where(row >= V_ANS, row + T, row)
            trow = jnp.right_shift(row, 1)
            cbase = jnp.left_shift(jnp.bitwise_and(row, 1), 6)
            for q in range(D // L):
                col = cbase + (lax.iota(jnp.int32, L) + (L * q))
                out_buf[r >> 1, pl.ds(((r & 1) << 6) + L * q, L)] = (
                    plsc.load_gather(table, [trow, col])
                )

        pltpu.async_copy(
            out_buf, out_hbm.at[pl.ds((b0 + i) * OCR_R, OCR_R)], sem_u
        )

        @pl.when(j < NPAIR - 1)
        def _():
            pltpu.async_copy(
                ocr_hbm.at[pl.ds((b0 + i + 2) * OCR_R, OCR_R)],
                table.at[slot_ds], sem_o,
            )

    def pair_step(j, carry):
        do_batch(j, 2 * j, 0, out0, so0, su0)
        do_batch(j, 2 * j + 1, 1, out1, so1, su1)
        return carry

    lax.fori_loop(0, NPAIR, pair_step, 0)
    pltpu.make_async_copy(out0, out_hbm.at[pl.ds(0, OCR_R)], su0).wait()
    pltpu.make_async_copy(out1, out_hbm.at[pl.ds(0, OCR_R)], su1).wait()


def kernel(ans_emb, ocr_emb, prev_inds):
    out = _gather_kernel(
        ans_emb.reshape(ANS_R, 128),
        ocr_emb.reshape(B * OCR_R, 128),
        prev_inds.astype(jnp.int32),
    )
    return out.reshape(B, T, D)
